# Initial kernel scaffold; baseline (speedup 1.0000x reference)
import functools

import jax
import jax.numpy as jnp
from jax import lax
from jax.experimental import pallas as pl
from jax.experimental.pallas import tpu as pltpu
from jax.experimental.pallas import tpu_sc as plsc

NN = 100000
EE = 1600000
LANE = 128
RPT = 391
EPAD = 32 * RPT * LANE
KROWS = 17
NCHUNK = 23
RN = 784
NP = RN * LANE

_mesh = plsc.VectorSubcoreMesh(core_axis_name="c", subcore_axis_name="s")


@functools.partial(
    pl.kernel,
    out_type=jax.ShapeDtypeStruct((2, NP), jnp.float32),
    mesh=_mesh,
    scratch_types=[
        pltpu.VMEM_SHARED((NP,), jnp.float32),
        pltpu.VMEM((KROWS, LANE), jnp.int32),
        pltpu.VMEM((LANE,), jnp.float32),
    ],
)
def _sc_degree(dst_hbm, zeros_hbm, out_hbm, acc, dbuf, ones):
    c = lax.axis_index("c")
    s = lax.axis_index("s")

    @pl.when(s == 0)
    def _():
        pltpu.sync_copy(zeros_hbm, acc)

    for t in range(LANE // 16):
        ones[pl.ds(t * 16, 16)] = jnp.full((16,), 1.0, jnp.float32)
    plsc.subcore_barrier()

    row0 = (s * 2 + c) * RPT

    def chunk(i, carry):
        r = row0 + i * KROWS
        pltpu.sync_copy(dst_hbm.at[pl.ds(r, KROWS)], dbuf)
        for j in range(KROWS):
            pltpu.sync_copy(ones, acc.at[dbuf.at[j]], add=True)
        return carry

    lax.fori_loop(0, NCHUNK, chunk, 0)
    plsc.subcore_barrier()

    @pl.when(s == 0)
    def _():
        pltpu.sync_copy(acc, out_hbm.at[c])


@functools.partial(
    pl.kernel,
    out_type=jax.ShapeDtypeStruct((2, NP), jnp.float32),
    mesh=_mesh,
    scratch_types=[
        pltpu.VMEM_SHARED((NP,), jnp.float32),
        pltpu.VMEM((KROWS, LANE), jnp.int32),
        pltpu.VMEM((KROWS, LANE), jnp.int32),
        pltpu.VMEM((KROWS, LANE), jnp.float32),
        pltpu.SemaphoreType.DMA,
    ],
)
def _sc_gather_scatter1(src_hbm, dst_hbm, y_hbm, zeros_hbm, out_hbm,
                        acc, sbuf, dbuf, vals, sem):
    c = lax.axis_index("c")
    s = lax.axis_index("s")

    @pl.when(s == 0)
    def _():
        pltpu.sync_copy(zeros_hbm, acc)

    plsc.subcore_barrier()
    row0 = (s * 2 + c) * RPT

    def chunk(i, carry):
        r = row0 + i * KROWS
        pltpu.sync_copy(src_hbm.at[pl.ds(r, KROWS)], sbuf)
        pltpu.sync_copy(dst_hbm.at[pl.ds(r, KROWS)], dbuf)
        for j in range(KROWS):
            pltpu.async_copy(y_hbm.at[sbuf.at[j]], vals.at[j], sem).wait()
        for j in range(KROWS):
            pltpu.sync_copy(vals.at[j], acc.at[dbuf.at[j]], add=True)
        return carry

    lax.fori_loop(0, NCHUNK, chunk, 0)
    plsc.subcore_barrier()

    @pl.when(s == 0)
    def _():
        pltpu.sync_copy(acc, out_hbm.at[c])


@functools.partial(
    pl.kernel,
    out_type=[
        jax.ShapeDtypeStruct((2, NP), jnp.float32),
        jax.ShapeDtypeStruct((2, NP), jnp.float32),
    ],
    mesh=_mesh,
    scratch_types=[
        pltpu.VMEM_SHARED((NP,), jnp.float32),
        pltpu.VMEM_SHARED((NP,), jnp.float32),
        pltpu.VMEM((KROWS, LANE), jnp.int32),
        pltpu.VMEM((KROWS, LANE), jnp.int32),
        pltpu.VMEM((KROWS, LANE), jnp.float32),
        pltpu.VMEM((KROWS, LANE), jnp.float32),
        pltpu.SemaphoreType.DMA,
    ],
)
def _sc_gather_scatter2(src_hbm, dst_hbm, yp_hbm, ym_hbm, zeros_hbm,
                        outp_hbm, outm_hbm,
                        accp, accm, sbuf, dbuf, valsp, valsm, sem):
    c = lax.axis_index("c")
    s = lax.axis_index("s")

    @pl.when(s == 0)
    def _():
        pltpu.sync_copy(zeros_hbm, accp)

    @pl.when(s == 1)
    def _():
        pltpu.sync_copy(zeros_hbm, accm)

    plsc.subcore_barrier()
    row0 = (s * 2 + c) * RPT

    def chunk(i, carry):
        r = row0 + i * KROWS
        pltpu.sync_copy(src_hbm.at[pl.ds(r, KROWS)], sbuf)
        pltpu.sync_copy(dst_hbm.at[pl.ds(r, KROWS)], dbuf)
        for j in range(KROWS):
            pltpu.async_copy(yp_hbm.at[sbuf.at[j]], valsp.at[j], sem).wait()
            pltpu.async_copy(ym_hbm.at[sbuf.at[j]], valsm.at[j], sem).wait()
        for j in range(KROWS):
            pltpu.sync_copy(valsp.at[j], accp.at[dbuf.at[j]], add=True)
            pltpu.sync_copy(valsm.at[j], accm.at[dbuf.at[j]], add=True)
        return carry

    lax.fori_loop(0, NCHUNK, chunk, 0)
    plsc.subcore_barrier()

    @pl.when(s == 0)
    def _():
        pltpu.sync_copy(accp, outp_hbm.at[c])

    @pl.when(s == 1)
    def _():
        pltpu.sync_copy(accm, outm_hbm.at[c])


def _tc1_body(degp_ref, x_ref, dinv_ref, y_ref):
    deg = degp_ref[0] + degp_ref[1] + 1.0
    dinv = lax.rsqrt(deg)
    dinv_ref[...] = dinv
    y_ref[...] = dinv * x_ref[...]


def _tc1(degp, xp):
    return pl.pallas_call(
        _tc1_body,
        out_shape=[
            jax.ShapeDtypeStruct((RN, LANE), jnp.float32),
            jax.ShapeDtypeStruct((RN, LANE), jnp.float32),
        ],
    )(degp, xp)


def _tc2_body(t1_ref, dinv_ref, x_ref, agg1_ref, yp_ref, ym_ref):
    dinv = dinv_ref[...]
    agg1 = dinv * (t1_ref[0] + t1_ref[1]) + dinv * dinv * x_ref[...]
    agg1_ref[...] = agg1
    z = dinv * agg1
    yp_ref[...] = jnp.maximum(z, 0.0)
    ym_ref[...] = jnp.maximum(-z, 0.0)


def _tc2(t1, dinv, xp):
    return pl.pallas_call(
        _tc2_body,
        out_shape=[
            jax.ShapeDtypeStruct((RN, LANE), jnp.float32),
            jax.ShapeDtypeStruct((RN, LANE), jnp.float32),
            jax.ShapeDtypeStruct((RN, LANE), jnp.float32),
        ],
    )(t1, dinv, xp)


def _tc3_body(tp_ref, tm_ref, dinv_ref, agg1_ref, w1_ref, w2_ref, b2_ref,
              wfc_ref, bfc_ref, out_ref):
    dinv = dinv_ref[...]
    d2 = dinv * dinv
    agg1 = agg1_ref[...]
    p = jnp.maximum(agg1, 0.0)
    m = jnp.maximum(-agg1, 0.0)
    P = dinv * (tp_ref[0] + tp_ref[1]) + d2 * p
    M = dinv * (tm_ref[0] + tm_ref[1]) + d2 * m
    w1 = w1_ref[...]
    w2 = w2_ref[...]
    u = jnp.sum(jnp.maximum(w1, 0.0).reshape(32, 1) * w2, axis=0,
                keepdims=True)
    v = jnp.sum(jnp.maximum(-w1, 0.0).reshape(32, 1) * w2, axis=0,
                keepdims=True)
    acc = jnp.zeros((RN, LANE), jnp.float32)
    for j in range(64):
        h = jnp.maximum(P * u[0, j] + M * v[0, j] + b2_ref[0, j], 0.0)
        acc = acc + h * wfc_ref[0, j]
    out_ref[...] = acc + bfc_ref[0, 0]


def _tc3(tp, tm, dinv, agg1, W1, W2, b2, Wfc, bfc):
    return pl.pallas_call(
        _tc3_body,
        out_shape=jax.ShapeDtypeStruct((RN, LANE), jnp.float32),
    )(tp, tm, dinv, agg1, W1, W2, b2.reshape(1, 64), Wfc.reshape(1, 64),
      bfc.reshape(1, 1))


def kernel(x_tmp, edge_index, W1, b1, W2, b2, Wfc, bfc):
    ei = edge_index.astype(jnp.int32)
    src = ei[0]
    dst = ei[1]
    src2d = jnp.concatenate(
        [src, jnp.zeros((EPAD - EE,), jnp.int32)]).reshape(-1, LANE)
    dst2d = jnp.concatenate(
        [dst, jnp.full((EPAD - EE,), NN, jnp.int32)]).reshape(-1, LANE)
    xp = jnp.concatenate(
        [x_tmp.reshape(-1), jnp.zeros((NP - NN,), jnp.float32)]
    ).reshape(RN, LANE)
    zeros = jnp.zeros((NP,), jnp.float32)

    degp = _sc_degree(dst2d, zeros)
    dinv, y = _tc1(degp.reshape(2, RN, LANE), xp)
    t1 = _sc_gather_scatter1(src2d, dst2d, y.reshape(NP), zeros)
    agg1, yp, ym = _tc2(t1.reshape(2, RN, LANE), dinv, xp)
    tp, tm = _sc_gather_scatter2(src2d, dst2d, yp.reshape(NP),
                                 ym.reshape(NP), zeros)
    out = _tc3(tp.reshape(2, RN, LANE), tm.reshape(2, RN, LANE),
               dinv, agg1, W1, W2, b2, Wfc, bfc)
    return out.reshape(NP)[:NN].reshape(NN, 1)


# speedup vs baseline: 33.0285x; 33.0285x over previous
import functools

import jax
import jax.numpy as jnp
from jax import lax
from jax.experimental import pallas as pl
from jax.experimental.pallas import tpu as pltpu
from jax.experimental.pallas import tpu_sc as plsc

NN = 100000
EE = 1600000
LANE = 128
RPT = 392
EPAD = 32 * RPT * LANE
KROWS = 8
NCHUNK = 49
RN = 784
NP = RN * LANE

_mesh = plsc.VectorSubcoreMesh(core_axis_name="c", subcore_axis_name="s")


@functools.partial(
    pl.kernel,
    out_type=jax.ShapeDtypeStruct((2 * NP,), jnp.float32),
    mesh=_mesh,
    scratch_types=[
        pltpu.VMEM_SHARED((NP,), jnp.float32),
        pltpu.VMEM((KROWS, LANE), jnp.int32),
        pltpu.VMEM((LANE,), jnp.float32),
    ],
)
def _sc_degree(dst_hbm, zeros_hbm, out_hbm, acc, dbuf, ones):
    c = lax.axis_index("c")
    s = lax.axis_index("s")

    @pl.when(s == 0)
    def _():
        pltpu.sync_copy(zeros_hbm, acc)

    for t in range(LANE // 16):
        ones[pl.ds(t * 16, 16)] = jnp.full((16,), 1.0, jnp.float32)
    plsc.subcore_barrier()

    row0 = (s * 2 + c) * RPT

    def chunk(i, carry):
        r = row0 + i * KROWS
        pltpu.sync_copy(dst_hbm.at[pl.ds(r, KROWS)], dbuf)
        for j in range(KROWS):
            pltpu.sync_copy(ones, acc.at[dbuf.at[j]], add=True)
        return carry

    lax.fori_loop(0, NCHUNK, chunk, 0)
    plsc.subcore_barrier()

    @pl.when(s == 0)
    def _():
        pltpu.sync_copy(acc, out_hbm.at[pl.ds(c * NP, NP)])


@functools.partial(
    pl.kernel,
    out_type=jax.ShapeDtypeStruct((2 * NP,), jnp.float32),
    mesh=_mesh,
    scratch_types=[
        pltpu.VMEM_SHARED((NP,), jnp.float32),
        pltpu.VMEM((KROWS, LANE), jnp.int32),
        pltpu.VMEM((KROWS, LANE), jnp.int32),
        pltpu.VMEM((KROWS, LANE), jnp.float32),
        pltpu.SemaphoreType.DMA,
    ],
)
def _sc_gather_scatter1(src_hbm, dst_hbm, y_hbm, zeros_hbm, out_hbm,
                        acc, sbuf, dbuf, vals, sem):
    c = lax.axis_index("c")
    s = lax.axis_index("s")

    @pl.when(s == 0)
    def _():
        pltpu.sync_copy(zeros_hbm, acc)

    plsc.subcore_barrier()
    row0 = (s * 2 + c) * RPT

    def chunk(i, carry):
        r = row0 + i * KROWS
        pltpu.sync_copy(src_hbm.at[pl.ds(r, KROWS)], sbuf)
        pltpu.sync_copy(dst_hbm.at[pl.ds(r, KROWS)], dbuf)
        for j in range(KROWS):
            pltpu.async_copy(y_hbm.at[sbuf.at[j]], vals.at[j], sem).wait()
        for j in range(KROWS):
            pltpu.sync_copy(vals.at[j], acc.at[dbuf.at[j]], add=True)
        return carry

    lax.fori_loop(0, NCHUNK, chunk, 0)
    plsc.subcore_barrier()

    @pl.when(s == 0)
    def _():
        pltpu.sync_copy(acc, out_hbm.at[pl.ds(c * NP, NP)])


@functools.partial(
    pl.kernel,
    out_type=[
        jax.ShapeDtypeStruct((2 * NP,), jnp.float32),
        jax.ShapeDtypeStruct((2 * NP,), jnp.float32),
    ],
    mesh=_mesh,
    scratch_types=[
        pltpu.VMEM_SHARED((NP,), jnp.float32),
        pltpu.VMEM_SHARED((NP,), jnp.float32),
        pltpu.VMEM((KROWS, LANE), jnp.int32),
        pltpu.VMEM((KROWS, LANE), jnp.int32),
        pltpu.VMEM((KROWS, LANE), jnp.float32),
        pltpu.VMEM((KROWS, LANE), jnp.float32),
        pltpu.SemaphoreType.DMA,
    ],
)
def _sc_gather_scatter2(src_hbm, dst_hbm, yp_hbm, ym_hbm, zeros_hbm,
                        outp_hbm, outm_hbm,
                        accp, accm, sbuf, dbuf, valsp, valsm, sem):
    c = lax.axis_index("c")
    s = lax.axis_index("s")

    @pl.when(s == 0)
    def _():
        pltpu.sync_copy(zeros_hbm, accp)

    @pl.when(s == 1)
    def _():
        pltpu.sync_copy(zeros_hbm, accm)

    plsc.subcore_barrier()
    row0 = (s * 2 + c) * RPT

    def chunk(i, carry):
        r = row0 + i * KROWS
        pltpu.sync_copy(src_hbm.at[pl.ds(r, KROWS)], sbuf)
        pltpu.sync_copy(dst_hbm.at[pl.ds(r, KROWS)], dbuf)
        for j in range(KROWS):
            pltpu.async_copy(yp_hbm.at[sbuf.at[j]], valsp.at[j], sem).wait()
            pltpu.async_copy(ym_hbm.at[sbuf.at[j]], valsm.at[j], sem).wait()
        for j in range(KROWS):
            pltpu.sync_copy(valsp.at[j], accp.at[dbuf.at[j]], add=True)
            pltpu.sync_copy(valsm.at[j], accm.at[dbuf.at[j]], add=True)
        return carry

    lax.fori_loop(0, NCHUNK, chunk, 0)
    plsc.subcore_barrier()

    @pl.when(s == 0)
    def _():
        pltpu.sync_copy(accp, outp_hbm.at[pl.ds(c * NP, NP)])

    @pl.when(s == 1)
    def _():
        pltpu.sync_copy(accm, outm_hbm.at[pl.ds(c * NP, NP)])


def _tc1_body(degp_ref, x_ref, dinv_ref, y_ref):
    deg = degp_ref[0] + degp_ref[1] + 1.0
    dinv = lax.rsqrt(deg)
    dinv_ref[...] = dinv
    y_ref[...] = dinv * x_ref[...]


def _tc1(degp, xp):
    return pl.pallas_call(
        _tc1_body,
        out_shape=[
            jax.ShapeDtypeStruct((RN, LANE), jnp.float32),
            jax.ShapeDtypeStruct((RN, LANE), jnp.float32),
        ],
    )(degp, xp)


def _tc2_body(t1_ref, dinv_ref, x_ref, agg1_ref, yp_ref, ym_ref):
    dinv = dinv_ref[...]
    agg1 = dinv * (t1_ref[0] + t1_ref[1]) + dinv * dinv * x_ref[...]
    agg1_ref[...] = agg1
    z = dinv * agg1
    yp_ref[...] = jnp.maximum(z, 0.0)
    ym_ref[...] = jnp.maximum(-z, 0.0)


def _tc2(t1, dinv, xp):
    return pl.pallas_call(
        _tc2_body,
        out_shape=[
            jax.ShapeDtypeStruct((RN, LANE), jnp.float32),
            jax.ShapeDtypeStruct((RN, LANE), jnp.float32),
            jax.ShapeDtypeStruct((RN, LANE), jnp.float32),
        ],
    )(t1, dinv, xp)


def _tc3_body(tp_ref, tm_ref, dinv_ref, agg1_ref, w1_ref, w2_ref, b2_ref,
              wfc_ref, bfc_ref, out_ref):
    dinv = dinv_ref[...]
    d2 = dinv * dinv
    agg1 = agg1_ref[...]
    p = jnp.maximum(agg1, 0.0)
    m = jnp.maximum(-agg1, 0.0)
    P = dinv * (tp_ref[0] + tp_ref[1]) + d2 * p
    M = dinv * (tm_ref[0] + tm_ref[1]) + d2 * m
    w1 = w1_ref[...]
    w2 = w2_ref[...]
    u = jnp.sum(jnp.maximum(w1, 0.0).reshape(32, 1) * w2, axis=0,
                keepdims=True)
    v = jnp.sum(jnp.maximum(-w1, 0.0).reshape(32, 1) * w2, axis=0,
                keepdims=True)
    acc = jnp.zeros((RN, LANE), jnp.float32)
    for j in range(64):
        h = jnp.maximum(P * u[0, j] + M * v[0, j] + b2_ref[0, j], 0.0)
        acc = acc + h * wfc_ref[0, j]
    out_ref[...] = acc + bfc_ref[0, 0]


def _tc3(tp, tm, dinv, agg1, W1, W2, b2, Wfc, bfc):
    return pl.pallas_call(
        _tc3_body,
        out_shape=jax.ShapeDtypeStruct((RN, LANE), jnp.float32),
    )(tp, tm, dinv, agg1, W1, W2, b2.reshape(1, 64), Wfc.reshape(1, 64),
      bfc.reshape(1, 1))


def kernel(x_tmp, edge_index, W1, b1, W2, b2, Wfc, bfc):
    ei = edge_index.astype(jnp.int32)
    src = ei[0]
    dst = ei[1]
    src2d = jnp.concatenate(
        [src, jnp.zeros((EPAD - EE,), jnp.int32)]).reshape(-1, LANE)
    dst2d = jnp.concatenate(
        [dst, jnp.full((EPAD - EE,), NN, jnp.int32)]).reshape(-1, LANE)
    xp = jnp.concatenate(
        [x_tmp.reshape(-1), jnp.zeros((NP - NN,), jnp.float32)]
    ).reshape(RN, LANE)
    zeros = jnp.zeros((NP,), jnp.float32)

    degp = _sc_degree(dst2d, zeros)
    dinv, y = _tc1(degp.reshape(2, RN, LANE), xp)
    t1 = _sc_gather_scatter1(src2d, dst2d, y.reshape(NP), zeros)
    agg1, yp, ym = _tc2(t1.reshape(2, RN, LANE), dinv, xp)
    tp, tm = _sc_gather_scatter2(src2d, dst2d, yp.reshape(NP),
                                 ym.reshape(NP), zeros)
    out = _tc3(tp.reshape(2, RN, LANE), tm.reshape(2, RN, LANE),
               dinv, agg1, W1, W2, b2, Wfc, bfc)
    return out.reshape(NP)[:NN].reshape(NN, 1)
